# baseline (speedup 1.0000x reference)
import functools

import jax
import jax.numpy as jnp
from jax import lax
from jax.experimental import pallas as pl
from jax.experimental.pallas import tpu as pltpu
from jax.experimental.pallas import tpu_sc as plsc


@functools.cache
def _make_sc_embed(B: int, S: int, V: int, D: int):
    info = plsc.get_sparse_core_info()
    NC, NS, L = info.num_cores, info.num_subcores, info.num_lanes
    NW = NC * NS
    assert S % NW == 0 and B % 2 == 0
    s_per_w = S // NW
    SUB = 16
    POSC = 2 * SUB
    n_pair = B // 2
    n_t = s_per_w // SUB
    n_groups = n_t * n_pair
    n_posc = s_per_w // POSC
    mesh = plsc.VectorSubcoreMesh(core_axis_name="c", subcore_axis_name="s")

    @functools.partial(
        pl.kernel,
        mesh=mesh,
        out_type=jax.ShapeDtypeStruct((B * S, D), jnp.float32),
        scratch_types=[
            pltpu.VMEM((B * s_per_w,), jnp.int32),
            pltpu.VMEM((n_groups * 2 * SUB,), jnp.int32),
            pltpu.VMEM((2 * SUB, D), jnp.float32),
            pltpu.VMEM((2 * SUB, D), jnp.float32),
            pltpu.VMEM((POSC, D), jnp.float32),
            pltpu.SemaphoreType.DMA,
            pltpu.SemaphoreType.DMA,
            pltpu.SemaphoreType.DMA,
            pltpu.SemaphoreType.DMA,
            pltpu.SemaphoreType.DMA,
        ],
    )
    def emb(idx_hbm, table_hbm, pos_hbm, out_hbm,
            idx_v, idx_v2, w0, w1, pos_v, g0, g1, o0, o1, psem):
        wid = lax.axis_index("s") * NC + lax.axis_index("c")
        s_base = wid * s_per_w
        wbuf = (w0, w1)
        gsem = (g0, g1)
        osem = (o0, o1)

        for b in range(B):
            pltpu.sync_copy(
                idx_hbm.at[pl.ds(b * S + s_base, s_per_w)],
                idx_v.at[pl.ds(b * s_per_w, s_per_w)],
            )

        for g in range(n_groups):
            t, p = divmod(g, n_pair)
            for i in range(2):
                v = idx_v[pl.ds((2 * p + i) * s_per_w + t * SUB, L)]
                idx_v2[pl.ds((g * 2 + i) * SUB, L)] = v

        def gather(g, buf):
            return pltpu.async_copy(
                table_hbm.at[idx_v2.at[pl.ds(g * 2 * SUB, 2 * SUB)]],
                wbuf[buf], gsem[buf],
            )

        def fill_pos(c):
            return pltpu.async_copy(
                pos_hbm.at[pl.ds(s_base + c * POSC, POSC)], pos_v, psem
            )

        def add_group(g, buf):
            t, _ = divmod(g, n_pair)
            rbase = (t % 2) * SUB
            cur = wbuf[buf]

            def body(r, carry):
                for j in range(D // L):
                    sl = pl.ds(j * L, L)
                    pv = pos_v[rbase + r, sl]
                    plsc.addupdate(cur.at[r, sl], pv)
                    plsc.addupdate(cur.at[SUB + r, sl], pv)
                return carry

            lax.fori_loop(0, SUB, body, 0)

        def store_group(g, buf):
            t, p = divmod(g, n_pair)
            return [pltpu.async_copy(
                wbuf[buf].at[pl.ds(i * SUB, SUB)],
                out_hbm.at[pl.ds((2 * p + i) * S + s_base + t * SUB, SUB)],
                osem[buf],
            ) for i in range(2)]

        pend_pos = fill_pos(0)
        pending_g = gather(0, 0)
        pending_o = [[], []]
        for g in range(n_groups):
            t, p = divmod(g, n_pair)
            cur = g % 2
            nxt = (g + 1) % 2
            if g % (2 * n_pair) == 0:
                pend_pos.wait()
            if g + 1 < n_groups:
                for d in pending_o[nxt]:
                    d.wait()
                pending_o[nxt] = []
                next_g = gather(g + 1, nxt)
            pending_g.wait()
            add_group(g, cur)
            pending_o[cur] = store_group(g, cur)
            if g % (2 * n_pair) == 2 * n_pair - 1 and g + 1 < n_groups:
                pend_pos = fill_pos(g // (2 * n_pair) + 1)
            if g + 1 < n_groups:
                pending_g = next_g
        for descs in pending_o:
            for d in descs:
                d.wait()

    return emb


def kernel(input_ids, word_embeddings, position_embeddings):
    B, S = input_ids.shape
    V, D = word_embeddings.shape
    ids_flat = input_ids.reshape(-1).astype(jnp.int32)
    emb = _make_sc_embed(B, S, V, D)
    out = emb(ids_flat, word_embeddings, position_embeddings)
    return out.reshape(B, S, D)


# speedup vs baseline: 1.3357x; 1.3357x over previous
import functools

import jax
import jax.numpy as jnp
from jax import lax
from jax.experimental import pallas as pl
from jax.experimental.pallas import tpu as pltpu
from jax.experimental.pallas import tpu_sc as plsc


@functools.cache
def _make_sc_embed(B: int, S: int, V: int, D: int):
    info = plsc.get_sparse_core_info()
    NC, NS, L = info.num_cores, info.num_subcores, info.num_lanes
    NW = NC * NS
    assert S % NW == 0 and B % 2 == 0
    s_per_w = S // NW
    SUB = 16
    POSC = 2 * SUB
    n_pair = B // 2
    n_t = s_per_w // SUB
    n_groups = n_t * n_pair
    n_posc = s_per_w // POSC
    mesh = plsc.VectorSubcoreMesh(core_axis_name="c", subcore_axis_name="s")

    @functools.partial(
        pl.kernel,
        mesh=mesh,
        out_type=jax.ShapeDtypeStruct((B * S, D), jnp.float32),
        scratch_types=[
            pltpu.VMEM((B * s_per_w,), jnp.int32),
            pltpu.VMEM((n_groups * 2 * SUB,), jnp.int32),
            pltpu.VMEM((2 * SUB, D), jnp.float32),
            pltpu.VMEM((2 * SUB, D), jnp.float32),
            pltpu.VMEM((POSC, D), jnp.float32),
            pltpu.SemaphoreType.DMA,
            pltpu.SemaphoreType.DMA,
            pltpu.SemaphoreType.DMA,
            pltpu.SemaphoreType.DMA,
            pltpu.SemaphoreType.DMA,
        ],
    )
    def emb(idx_hbm, table_hbm, pos_hbm, out_hbm,
            idx_v, idx_v2, w0, w1, pos_v, g0, g1, o0, o1, psem):
        wid = lax.axis_index("s") * NC + lax.axis_index("c")
        s_base = wid * s_per_w
        wbuf = (w0, w1)
        gsem = (g0, g1)
        osem = (o0, o1)

        for b in range(B):
            pltpu.sync_copy(
                idx_hbm.at[pl.ds(b * S + s_base, s_per_w)],
                idx_v.at[pl.ds(b * s_per_w, s_per_w)],
            )

        for g in range(n_groups):
            t, p = divmod(g, n_pair)
            for i in range(2):
                v = idx_v[pl.ds((2 * p + i) * s_per_w + t * SUB, L)]
                idx_v2[pl.ds((g * 2 + i) * SUB, L)] = v

        def gather(g, buf):
            return pltpu.async_copy(
                table_hbm.at[idx_v2.at[pl.ds(g * 2 * SUB, 2 * SUB)]],
                wbuf[buf], gsem[buf],
            )

        def fill_pos(c):
            return pltpu.async_copy(
                pos_hbm.at[pl.ds(s_base + c * POSC, POSC)], pos_v, psem
            )

        def add_group(g, buf):
            t, _ = divmod(g, n_pair)
            rbase = (t % 2) * SUB
            cur = wbuf[buf]

            def body(r, carry):
                for j0 in range(0, D // L, 4):
                    sls = [pl.ds((j0 + u) * L, L) for u in range(4)]
                    pvs = [pos_v[rbase + r, sl] for sl in sls]
                    for u in range(4):
                        plsc.addupdate(cur.at[r, sls[u]], pvs[u])
                        plsc.addupdate(cur.at[SUB + r, sls[u]], pvs[u])
                return carry

            lax.fori_loop(0, SUB, body, 0)

        def store_group(g, buf):
            t, p = divmod(g, n_pair)
            return [pltpu.async_copy(
                wbuf[buf].at[pl.ds(i * SUB, SUB)],
                out_hbm.at[pl.ds((2 * p + i) * S + s_base + t * SUB, SUB)],
                osem[buf],
            ) for i in range(2)]

        pend_pos = fill_pos(0)
        pending_g = gather(0, 0)
        pending_o = [[], []]
        for g in range(n_groups):
            t, p = divmod(g, n_pair)
            cur = g % 2
            nxt = (g + 1) % 2
            if g % (2 * n_pair) == 0:
                pend_pos.wait()
            if g + 1 < n_groups:
                for d in pending_o[nxt]:
                    d.wait()
                pending_o[nxt] = []
                next_g = gather(g + 1, nxt)
            pending_g.wait()
            add_group(g, cur)
            pending_o[cur] = store_group(g, cur)
            if g % (2 * n_pair) == 2 * n_pair - 1 and g + 1 < n_groups:
                pend_pos = fill_pos(g // (2 * n_pair) + 1)
            if g + 1 < n_groups:
                pending_g = next_g
        for descs in pending_o:
            for d in descs:
                d.wait()

    return emb


def kernel(input_ids, word_embeddings, position_embeddings):
    B, S = input_ids.shape
    V, D = word_embeddings.shape
    ids_flat = input_ids.reshape(-1).astype(jnp.int32)
    emb = _make_sc_embed(B, S, V, D)
    out = emb(ids_flat, word_embeddings, position_embeddings)
    return out.reshape(B, S, D)


# speedup vs baseline: 1.5417x; 1.1542x over previous
import functools

import jax
import jax.numpy as jnp
from jax import lax
from jax.experimental import pallas as pl
from jax.experimental.pallas import tpu as pltpu
from jax.experimental.pallas import tpu_sc as plsc


@functools.cache
def _make_sc_embed(B: int, S: int, V: int, D: int):
    info = plsc.get_sparse_core_info()
    NC, NS, L = info.num_cores, info.num_subcores, info.num_lanes
    NW = NC * NS
    assert S % NW == 0
    s_per_w = S // NW
    SUB = 8
    n_t = s_per_w // SUB
    GROW = B * SUB
    POSC = 4 * SUB
    n_posc = s_per_w // POSC
    mesh = plsc.VectorSubcoreMesh(core_axis_name="c", subcore_axis_name="s")

    @functools.partial(
        pl.kernel,
        mesh=mesh,
        out_type=jax.ShapeDtypeStruct((B * S, D), jnp.float32),
        scratch_types=[
            pltpu.VMEM((B * s_per_w,), jnp.int32),
            pltpu.VMEM((GROW, D), jnp.float32),
            pltpu.VMEM((GROW, D), jnp.float32),
            pltpu.VMEM((POSC, D), jnp.float32),
            pltpu.SemaphoreType.DMA,
            pltpu.SemaphoreType.DMA,
            pltpu.SemaphoreType.DMA,
            pltpu.SemaphoreType.DMA,
            pltpu.SemaphoreType.DMA,
            pltpu.SemaphoreType.DMA,
        ],
    )
    def emb(idx_hbm, table_hbm, pos_hbm, out_hbm,
            idx_v, w0, w1, pos_v, g0, g1, o0, o1, psem, isem):
        wid = lax.axis_index("s") * NC + lax.axis_index("c")
        s_base = wid * s_per_w
        wbuf = (w0, w1)
        gsem = (g0, g1)
        osem = (o0, o1)

        pltpu.async_copy(
            idx_hbm.at[pl.ds(wid * B * s_per_w, B * s_per_w)], idx_v, isem
        ).wait()

        def gather(t, buf):
            return pltpu.async_copy(
                table_hbm.at[idx_v.at[pl.ds(t * GROW, GROW)]],
                wbuf[buf], gsem[buf],
            )

        def fill_pos(c):
            return pltpu.async_copy(
                pos_hbm.at[pl.ds(s_base + c * POSC, POSC)], pos_v, psem
            )

        def add_group(t, buf):
            rbase = (t % 4) * SUB
            cur = wbuf[buf]

            def body(r, carry):
                def jbody(jq, c2):
                    base = pl.multiple_of(jq * 4 * L, 4 * L)
                    sls = [pl.ds(base + u * L, L) for u in range(4)]
                    pvs = [pos_v[rbase + r, sl] for sl in sls]
                    for u in range(4):
                        for b in range(B):
                            plsc.addupdate(cur.at[b * SUB + r, sls[u]], pvs[u])
                    return c2

                return lax.fori_loop(0, D // L // 4, jbody, carry)

            lax.fori_loop(0, SUB, body, 0)

        def store_group(t, buf):
            return [pltpu.async_copy(
                wbuf[buf].at[pl.ds(b * SUB, SUB)],
                out_hbm.at[pl.ds(b * S + s_base + t * SUB, SUB)],
                osem[buf],
            ) for b in range(B)]

        pend_pos = fill_pos(0)
        pending_g = gather(0, 0)
        pending_o = [[], []]
        for t in range(n_t):
            cur = t % 2
            nxt = (t + 1) % 2
            if t % 4 == 0:
                pend_pos.wait()
            if t + 1 < n_t:
                for d in pending_o[nxt]:
                    d.wait()
                pending_o[nxt] = []
                next_g = gather(t + 1, nxt)
            pending_g.wait()
            add_group(t, cur)
            pending_o[cur] = store_group(t, cur)
            if t % 4 == 3 and t + 1 < n_t:
                pend_pos = fill_pos(t // 4 + 1)
            if t + 1 < n_t:
                pending_g = next_g
        for descs in pending_o:
            for d in descs:
                d.wait()

    return emb


def kernel(input_ids, word_embeddings, position_embeddings):
    B, S = input_ids.shape
    V, D = word_embeddings.shape
    SUB = 8
    ids_grouped = (
        input_ids.astype(jnp.int32)
        .reshape(B, S // SUB, SUB)
        .transpose(1, 0, 2)
        .reshape(-1)
    )
    emb = _make_sc_embed(B, S, V, D)
    out = emb(ids_grouped, word_embeddings, position_embeddings)
    return out.reshape(B, S, D)
